# baseline (speedup 1.0000x reference)
import jax
import jax.numpy as jnp
from jax import lax
from jax.experimental import pallas as pl
from jax.experimental.pallas import tpu as pltpu
from jax.experimental.pallas import tpu_sc as plsc

_R = 64
_A = 16
_S = 4096
_L = 16
_NW = 32

_S_SC = 512
_S_TC = _S - _S_SC
_SPW = _S_SC // _NW
_NG = _SPW // _L


def _fls_sc_body(x_hbm, frb_hbm, c1_hbm, out_hbm, x_v, frb_v, c1_v, w_v,
                 out_v, num_v, den_v):
    cid = lax.axis_index("c")
    sid = lax.axis_index("s")
    wid = sid * 2 + cid
    base = wid * _SPW

    pltpu.sync_copy(x_hbm.at[wid], x_v)
    pltpu.sync_copy(frb_hbm, frb_v)
    pltpu.sync_copy(c1_hbm, c1_v.at[pl.ds(0, _R)])

    @plsc.parallel_loop(0, _R * _A // _L)
    def _pre(i):
        sig = frb_v[pl.ds(i * _L, _L)]
        w_v[pl.ds(i * _L, _L)] = -0.5 / (sig * sig)

    zeros = jnp.zeros((_L,), jnp.float32)

    @plsc.parallel_loop(0, _NG)
    def _zero(g):
        num_v[pl.ds(g * _L, _L)] = zeros
        den_v[pl.ds(g * _L, _L)] = zeros

    def _rule(r, carry):
        roff = r * _A
        wv = w_v[pl.ds(roff, _L)]
        cv = frb_v[pl.ds(roff + 1, _L)]
        ws = [wv[a] for a in range(_A)]
        cs = [cv[a] for a in range(_A)]
        c1r = c1_v[pl.ds(r, _L)][0]

        @plsc.parallel_loop(0, _NG, unroll=min(2, _NG))
        def _grp(g):
            q = zeros
            for a in range(_A):
                d = x_v[pl.ds(a * _SPW + g * _L, _L)] - cs[a]
                q = q + (ws[a] * d) * d
            uu = jnp.exp(q)
            num_v[pl.ds(g * _L, _L)] += c1r * uu
            den_v[pl.ds(g * _L, _L)] += uu

        return carry

    lax.fori_loop(0, _R, _rule, 0, unroll=1)

    for g in range(_NG):
        out_v[pl.ds(g * _L, _L)] = (
            num_v[pl.ds(g * _L, _L)] / den_v[pl.ds(g * _L, _L)])

    pltpu.sync_copy(out_v, out_hbm.at[pl.ds(base, _SPW)])


def _fls_tc_body(x_ref, sig_ref, c_ref, c1_ref, o_ref):
    x = x_ref[...]
    sig = sig_ref[...]
    c = c_ref[...]
    c1 = c1_ref[...]
    w = -0.5 / (sig * sig)
    g = (-2.0 * w) * c
    k = jnp.sum((w * c) * c, axis=1)
    bmat = jnp.concatenate([w, g], axis=1)
    amat = jnp.concatenate([x * x, x], axis=1)
    q = jax.lax.dot_general(
        amat, bmat, (((1,), (1,)), ((), ())),
        preferred_element_type=jnp.float32)
    uu = jnp.exp(q + k[None, :])
    num = jnp.sum(uu * c1, axis=1, keepdims=True)
    den = jnp.sum(uu, axis=1, keepdims=True)
    o_ref[...] = num / den


@jax.jit
def kernel(input_data, FRB_weights, c1):
    sig2d = lax.slice(FRB_weights, (0,), (_R * _A,)).reshape(_R, _A)
    c2d = lax.slice(FRB_weights, (1,), (_R * _A + 1,)).reshape(_R, _A)

    x_t = (input_data[_S_TC:].T.reshape(_A, _NW, _SPW)
           .transpose(1, 0, 2).reshape(_NW, _A * _SPW))
    run_sc = pl.kernel(
        _fls_sc_body,
        out_type=jax.ShapeDtypeStruct((_S_SC,), jnp.float32),
        mesh=plsc.VectorSubcoreMesh(core_axis_name="c", subcore_axis_name="s"),
        scratch_types=[
            pltpu.VMEM((_A * _SPW,), jnp.float32),
            pltpu.VMEM((2 * _R * _A,), jnp.float32),
            pltpu.VMEM((_R + _L,), jnp.float32),
            pltpu.VMEM((_R * _A,), jnp.float32),
            pltpu.VMEM((_SPW,), jnp.float32),
            pltpu.VMEM((_SPW,), jnp.float32),
            pltpu.VMEM((_SPW,), jnp.float32),
        ],
    )
    out_sc = run_sc(x_t, FRB_weights, c1)

    out_tc = pl.pallas_call(
        _fls_tc_body,
        out_shape=jax.ShapeDtypeStruct((_S_TC, 1), jnp.float32),
    )(input_data[:_S_TC], sig2d, c2d, c1.reshape(1, _R))

    return jnp.concatenate([out_tc.reshape(_S_TC), out_sc])


# speedup vs baseline: 1.0093x; 1.0093x over previous
import jax
import jax.numpy as jnp
from jax import lax
from jax.experimental import pallas as pl
from jax.experimental.pallas import tpu as pltpu
from jax.experimental.pallas import tpu_sc as plsc

_R = 64
_A = 16
_S = 4096
_L = 16
_NW = 32

_S_SC = 1024
_S_TC = _S - _S_SC
_SPW = _S_SC // _NW
_NG = _SPW // _L


def _fls_sc_body(x_hbm, frb_hbm, c1_hbm, out_hbm, x_v, frb_v, c1_v, w_v,
                 out_v, num_v, den_v):
    cid = lax.axis_index("c")
    sid = lax.axis_index("s")
    wid = sid * 2 + cid
    base = wid * _SPW

    pltpu.sync_copy(x_hbm.at[wid], x_v)
    pltpu.sync_copy(frb_hbm, frb_v)
    pltpu.sync_copy(c1_hbm, c1_v.at[pl.ds(0, _R)])

    @plsc.parallel_loop(0, _R * _A // _L)
    def _pre(i):
        sig = frb_v[pl.ds(i * _L, _L)]
        w_v[pl.ds(i * _L, _L)] = -0.5 / (sig * sig)

    zeros = jnp.zeros((_L,), jnp.float32)

    @plsc.parallel_loop(0, _NG)
    def _zero(g):
        num_v[pl.ds(g * _L, _L)] = zeros
        den_v[pl.ds(g * _L, _L)] = zeros

    def _rule(r, carry):
        roff = r * _A
        wv = w_v[pl.ds(roff, _L)]
        cv = frb_v[pl.ds(roff + 1, _L)]
        ws = [wv[a] for a in range(_A)]
        cs = [cv[a] for a in range(_A)]
        c1r = c1_v[pl.ds(r, _L)][0]

        @plsc.parallel_loop(0, _NG, unroll=min(2, _NG))
        def _grp(g):
            q = zeros
            for a in range(_A):
                d = x_v[pl.ds(a * _SPW + g * _L, _L)] - cs[a]
                q = q + (ws[a] * d) * d
            uu = jnp.exp(q)
            num_v[pl.ds(g * _L, _L)] += c1r * uu
            den_v[pl.ds(g * _L, _L)] += uu

        return carry

    lax.fori_loop(0, _R, _rule, 0, unroll=1)

    for g in range(_NG):
        out_v[pl.ds(g * _L, _L)] = (
            num_v[pl.ds(g * _L, _L)] / den_v[pl.ds(g * _L, _L)])

    pltpu.sync_copy(out_v, out_hbm.at[pl.ds(base, _SPW)])


def _fls_tc_body(x_ref, sig_ref, c_ref, c1_ref, o_ref):
    x = x_ref[...]
    sig = sig_ref[...]
    c = c_ref[...]
    c1 = c1_ref[...]
    w = -0.5 / (sig * sig)
    g = (-2.0 * w) * c
    k = jnp.sum((w * c) * c, axis=1)
    bmat = jnp.concatenate([w, g], axis=1)
    amat = jnp.concatenate([x * x, x], axis=1)
    q = jax.lax.dot_general(
        amat, bmat, (((1,), (1,)), ((), ())),
        preferred_element_type=jnp.float32)
    uu = jnp.exp(q + k[None, :])
    num = jnp.sum(uu * c1, axis=1, keepdims=True)
    den = jnp.sum(uu, axis=1, keepdims=True)
    o_ref[0:_S_TC, :] = num / den


@jax.jit
def kernel(input_data, FRB_weights, c1):
    sig2d = lax.slice(FRB_weights, (0,), (_R * _A,)).reshape(_R, _A)
    c2d = lax.slice(FRB_weights, (1,), (_R * _A + 1,)).reshape(_R, _A)

    x_t = (input_data[_S_TC:].T.reshape(_A, _NW, _SPW)
           .transpose(1, 0, 2).reshape(_NW, _A * _SPW))
    run_sc = pl.kernel(
        _fls_sc_body,
        out_type=jax.ShapeDtypeStruct((_S_SC,), jnp.float32),
        mesh=plsc.VectorSubcoreMesh(core_axis_name="c", subcore_axis_name="s"),
        scratch_types=[
            pltpu.VMEM((_A * _SPW,), jnp.float32),
            pltpu.VMEM((2 * _R * _A,), jnp.float32),
            pltpu.VMEM((_R + _L,), jnp.float32),
            pltpu.VMEM((_R * _A,), jnp.float32),
            pltpu.VMEM((_SPW,), jnp.float32),
            pltpu.VMEM((_SPW,), jnp.float32),
            pltpu.VMEM((_SPW,), jnp.float32),
        ],
    )
    out_sc = run_sc(x_t, FRB_weights, c1)

    out_tc = pl.pallas_call(
        _fls_tc_body,
        out_shape=jax.ShapeDtypeStruct((_S, 1), jnp.float32),
    )(input_data[:_S_TC], sig2d, c2d, c1.reshape(1, _R))

    return lax.dynamic_update_slice(out_tc.reshape(_S), out_sc, (_S_TC,))


# speedup vs baseline: 1.0470x; 1.0373x over previous
import jax
import jax.numpy as jnp
from jax import lax
from jax.experimental import pallas as pl
from jax.experimental.pallas import tpu as pltpu
from jax.experimental.pallas import tpu_sc as plsc

_R = 64
_A = 16
_S = 4096
_L = 16
_NW = 32

_S_SC = 1024
_S_TC = _S - _S_SC
_SPW = _S_SC // _NW
_NG = _SPW // _L


def _fls_sc_body(x_hbm, frb_hbm, c1_hbm, out_hbm, x_v, frb_v, c1_v, w_v,
                 out_v, num_v, den_v):
    cid = lax.axis_index("c")
    sid = lax.axis_index("s")
    wid = sid * 2 + cid
    base = wid * _SPW

    pltpu.sync_copy(x_hbm.at[wid], x_v)
    pltpu.sync_copy(frb_hbm, frb_v)
    pltpu.sync_copy(c1_hbm, c1_v.at[pl.ds(0, _R)])

    @plsc.parallel_loop(0, _R * _A // _L)
    def _pre(i):
        sig = frb_v[pl.ds(i * _L, _L)]
        w_v[pl.ds(i * _L, _L)] = -0.5 / (sig * sig)

    zeros = jnp.zeros((_L,), jnp.float32)

    @plsc.parallel_loop(0, _NG)
    def _zero(g):
        num_v[pl.ds(g * _L, _L)] = zeros
        den_v[pl.ds(g * _L, _L)] = zeros

    def _rule(r, carry):
        roff = r * _A
        wv = w_v[pl.ds(roff, _L)]
        cv = frb_v[pl.ds(roff + 1, _L)]
        ws = [wv[a] for a in range(_A)]
        cs = [cv[a] for a in range(_A)]
        c1r = c1_v[pl.ds(r, _L)][0]

        @plsc.parallel_loop(0, _NG, unroll=min(2, _NG))
        def _grp(g):
            q = zeros
            for a in range(_A):
                d = x_v[pl.ds(a * _SPW + g * _L, _L)] - cs[a]
                q = q + (ws[a] * d) * d
            uu = jnp.exp(q)
            num_v[pl.ds(g * _L, _L)] += c1r * uu
            den_v[pl.ds(g * _L, _L)] += uu

        return carry

    lax.fori_loop(0, _R, _rule, 0, unroll=1)

    for g in range(_NG):
        out_v[pl.ds(g * _L, _L)] = (
            num_v[pl.ds(g * _L, _L)] / den_v[pl.ds(g * _L, _L)])

    pltpu.sync_copy(out_v, out_hbm.at[pl.ds(base, _SPW)])


def _fls_tc_body(x_ref, sig_ref, c_ref, c1_ref, o_ref):
    x = x_ref[...]
    sig = sig_ref[...]
    c = c_ref[...]
    c1 = c1_ref[...]
    w = -0.5 / (sig * sig)
    g = (-2.0 * w) * c
    k = jnp.sum((w * c) * c, axis=1)
    bmat = jnp.concatenate([w, g], axis=1)
    amat = jnp.concatenate([x * x, x], axis=1)
    q = jax.lax.dot_general(
        amat, bmat, (((1,), (1,)), ((), ())),
        preferred_element_type=jnp.float32)
    uu = jnp.exp(q + k[None, :])
    num = jnp.sum(uu * c1, axis=1, keepdims=True)
    den = jnp.sum(uu, axis=1, keepdims=True)
    o_ref[...] = num / den


@jax.jit
def kernel(input_data, FRB_weights, c1):
    sig2d = lax.slice(FRB_weights, (0,), (_R * _A,)).reshape(_R, _A)
    c2d = lax.slice(FRB_weights, (1,), (_R * _A + 1,)).reshape(_R, _A)

    x_t = (input_data[_S_TC:].reshape(_NW, _SPW, _A)
           .transpose(0, 2, 1).reshape(_NW, _A * _SPW))
    run_sc = pl.kernel(
        _fls_sc_body,
        out_type=jax.ShapeDtypeStruct((_S_SC,), jnp.float32),
        mesh=plsc.VectorSubcoreMesh(core_axis_name="c", subcore_axis_name="s"),
        scratch_types=[
            pltpu.VMEM((_A * _SPW,), jnp.float32),
            pltpu.VMEM((2 * _R * _A,), jnp.float32),
            pltpu.VMEM((_R + _L,), jnp.float32),
            pltpu.VMEM((_R * _A,), jnp.float32),
            pltpu.VMEM((_SPW,), jnp.float32),
            pltpu.VMEM((_SPW,), jnp.float32),
            pltpu.VMEM((_SPW,), jnp.float32),
        ],
    )
    out_sc = run_sc(x_t, FRB_weights, c1)

    out_tc = pl.pallas_call(
        _fls_tc_body,
        out_shape=jax.ShapeDtypeStruct((_S_TC, 1), jnp.float32),
    )(input_data[:_S_TC], sig2d, c2d, c1.reshape(1, _R))

    return jnp.concatenate([out_tc.reshape(_S_TC), out_sc])
